# Initial kernel scaffold; baseline (speedup 1.0000x reference)
import jax
import jax.numpy as jnp
from jax.experimental import pallas as pl


def kernel(x, batch, W1, b1, W2, b2):
    raise NotImplementedError("write your pallas kernel here")


# speedup vs baseline: 1.8348x; 1.8348x over previous
import jax
import jax.numpy as jnp
from jax import lax
from jax.experimental import pallas as pl
from jax.experimental.pallas import tpu as pltpu
from jax.experimental.pallas import tpu_sc as plsc

N = 50000
D = 256
NSEG = 64

NPAD = 50176
BLK = 1024
GRID = NPAD // BLK

NW = 32
PER_W = NPAD // NW
CH = 112
NCH = PER_W // CH


def _score_body(x_ref, b3_ref, W1_ref, b1_ref, W2_ref, b2_ref, e_ref,
                den_ref):
    i = pl.program_id(0)
    x = x_ref[...]
    h = jnp.tanh(jnp.dot(x, W1_ref[...], preferred_element_type=jnp.float32)
                 + b1_ref[...])
    s = jnp.sum(h * W2_ref[...], axis=1) + b2_ref[0, 0]
    row = jax.lax.broadcasted_iota(jnp.int32, (BLK,), 0) + i * BLK
    e = jnp.where(row < N, jnp.exp(s), 0.0)
    e_ref[...] = e
    bb = b3_ref[0, 0, :]
    seg = jax.lax.broadcasted_iota(jnp.int32, (NSEG, BLK), 0)
    part = jnp.sum(jnp.where(bb[None, :] == seg, e[None, :], 0.0), axis=1)

    @pl.when(i == 0)
    def _init():
        den_ref[...] = jnp.zeros_like(den_ref)

    den_ref[...] = den_ref[...] + part[:, None]


def _attn_body(e_hbm, bidx_hbm, den_hbm, attn_hbm, e_v, i_v, a_v, d_v, sem):
    cid = lax.axis_index("c")
    sid = lax.axis_index("s")
    wid = cid * 16 + sid
    base = wid * PER_W
    for c in range(NCH):
        pltpu.sync_copy(e_hbm.at[pl.ds(base + c * CH, CH)], e_v.at[c])
        pltpu.sync_copy(bidx_hbm.at[pl.ds(base + c * CH, CH)], i_v.at[c])
    for c in range(NCH):
        pltpu.async_copy(den_hbm.at[i_v.at[c]], d_v.at[c], sem).wait()
    for c in range(NCH):
        for j in range(CH // 16):
            sl = pl.ds(j * 16, 16)
            a_v[c, sl] = e_v[c, sl] / (d_v[c, sl] + 1e-16)
    for c in range(NCH):
        pltpu.sync_copy(a_v.at[c], attn_hbm.at[pl.ds(base + c * CH, CH)])


def _pool_body(x_ref, a_ref, b3_ref, o_ref):
    i = pl.program_id(0)
    a = a_ref[...]
    bb = b3_ref[0, 0, :]
    seg = jax.lax.broadcasted_iota(jnp.int32, (NSEG, BLK), 0)
    m = jnp.where(bb[None, :] == seg, a[None, :], 0.0)

    @pl.when(i == 0)
    def _init():
        o_ref[...] = jnp.zeros_like(o_ref)

    o_ref[...] = o_ref[...] + jnp.dot(
        m, x_ref[...], preferred_element_type=jnp.float32)


def kernel(x, batch, W1, b1, W2, b2):
    batch32 = batch.astype(jnp.int32)
    bpad = jnp.pad(batch32, (0, NPAD - N))
    b3 = bpad.reshape(GRID, 1, BLK)
    b1r = b1.reshape(1, D)
    W2r = W2.reshape(1, D)
    b2r = b2.reshape(1, 1)

    e1d, den = pl.pallas_call(
        _score_body,
        grid=(GRID,),
        in_specs=[
            pl.BlockSpec((BLK, D), lambda i: (i, 0)),
            pl.BlockSpec((1, 1, BLK), lambda i: (i, 0, 0)),
            pl.BlockSpec((D, D), lambda i: (0, 0)),
            pl.BlockSpec((1, D), lambda i: (0, 0)),
            pl.BlockSpec((1, D), lambda i: (0, 0)),
            pl.BlockSpec((1, 1), lambda i: (0, 0)),
        ],
        out_specs=[
            pl.BlockSpec((BLK,), lambda i: (i,)),
            pl.BlockSpec((NSEG, 128), lambda i: (0, 0)),
        ],
        out_shape=[
            jax.ShapeDtypeStruct((NPAD,), jnp.float32),
            jax.ShapeDtypeStruct((NSEG, 128), jnp.float32),
        ],
    )(x, b3, W1, b1r, W2r, b2r)

    mesh = plsc.VectorSubcoreMesh(core_axis_name="c", subcore_axis_name="s",
                                  num_cores=2, num_subcores=16)
    attn = pl.kernel(
        _attn_body,
        out_type=jax.ShapeDtypeStruct((NPAD,), jnp.float32),
        mesh=mesh,
        scratch_types=[
            pltpu.VMEM((NCH, CH), jnp.float32),
            pltpu.VMEM((NCH, CH), jnp.int32),
            pltpu.VMEM((NCH, CH), jnp.float32),
            pltpu.VMEM((NCH, CH), jnp.float32),
            pltpu.SemaphoreType.DMA,
        ],
    )(e1d, bpad, den[:, 0])

    out = pl.pallas_call(
        _pool_body,
        grid=(GRID,),
        in_specs=[
            pl.BlockSpec((BLK, D), lambda i: (i, 0)),
            pl.BlockSpec((BLK,), lambda i: (i,)),
            pl.BlockSpec((1, 1, BLK), lambda i: (i, 0, 0)),
        ],
        out_specs=pl.BlockSpec((NSEG, D), lambda i: (0, 0)),
        out_shape=jax.ShapeDtypeStruct((NSEG, D), jnp.float32),
    )(x, attn, b3)
    return out


# speedup vs baseline: 5.9375x; 3.2361x over previous
import jax
import jax.numpy as jnp
from jax import lax
from jax.experimental import pallas as pl
from jax.experimental.pallas import tpu as pltpu
from jax.experimental.pallas import tpu_sc as plsc

N = 50000
D = 256
NSEG = 64

NPAD = 50176
BLK = 1024
GRID = NPAD // BLK

NW = 32
PER_W = NPAD // NW
NVEC = PER_W // 16


def _score_body(x_ref, b3_ref, W1_ref, b1_ref, W2_ref, b2_ref, e_ref,
                den_ref):
    i = pl.program_id(0)
    x = x_ref[...]
    h = jnp.tanh(jnp.dot(x, W1_ref[...], preferred_element_type=jnp.float32)
                 + b1_ref[...])
    s = jnp.sum(h * W2_ref[...], axis=1) + b2_ref[0, 0]
    row = jax.lax.broadcasted_iota(jnp.int32, (BLK,), 0) + i * BLK
    e = jnp.where(row < N, jnp.exp(s), 0.0)
    e_ref[...] = e
    bb = b3_ref[0, 0, :]
    seg = jax.lax.broadcasted_iota(jnp.int32, (NSEG, BLK), 0)
    part = jnp.sum(jnp.where(bb[None, :] == seg, e[None, :], 0.0), axis=1)

    @pl.when(i == 0)
    def _init():
        den_ref[...] = jnp.zeros_like(den_ref)

    den_ref[...] = den_ref[...] + part[:, None]


def _dyn_gather(vec, idx):
    return lax.gather(
        vec, idx[:, None],
        dimension_numbers=lax.GatherDimensionNumbers(
            offset_dims=(), collapsed_slice_dims=(0,), start_index_map=(0,)),
        slice_sizes=(1,),
        mode=lax.GatherScatterMode.PROMISE_IN_BOUNDS)


def _attn_body(e_hbm, bidx_hbm, den_hbm, attn_hbm, e_v, i_v, a_v, den_v):
    cid = lax.axis_index("c")
    sid = lax.axis_index("s")
    wid = cid * 16 + sid
    base = wid * PER_W
    pltpu.sync_copy(den_hbm, den_v)
    pltpu.sync_copy(e_hbm.at[pl.ds(base, PER_W)], e_v)
    pltpu.sync_copy(bidx_hbm.at[pl.ds(base, PER_W)], i_v)
    g0 = den_v[pl.ds(0, 16)]
    g1 = den_v[pl.ds(16, 16)]
    g2 = den_v[pl.ds(32, 16)]
    g3 = den_v[pl.ds(48, 16)]
    for j in range(NVEC):
        sl = pl.ds(j * 16, 16)
        idx = i_v[sl]
        lane = idx & 15
        d01 = jnp.where(idx < 16,
                        _dyn_gather(g0, lane),
                        _dyn_gather(g1, lane))
        d23 = jnp.where(idx < 48,
                        _dyn_gather(g2, lane),
                        _dyn_gather(g3, lane))
        d = jnp.where(idx < 32, d01, d23)
        a_v[sl] = e_v[sl] / (d + 1e-16)
    pltpu.sync_copy(a_v, attn_hbm.at[pl.ds(base, PER_W)])


def _pool_body(x_ref, a_ref, b3_ref, o_ref):
    i = pl.program_id(0)
    a = a_ref[...]
    bb = b3_ref[0, 0, :]
    seg = jax.lax.broadcasted_iota(jnp.int32, (NSEG, BLK), 0)
    m = jnp.where(bb[None, :] == seg, a[None, :], 0.0)

    @pl.when(i == 0)
    def _init():
        o_ref[...] = jnp.zeros_like(o_ref)

    o_ref[...] = o_ref[...] + jnp.dot(
        m, x_ref[...], preferred_element_type=jnp.float32)


def kernel(x, batch, W1, b1, W2, b2):
    batch32 = batch.astype(jnp.int32)
    bpad = jnp.pad(batch32, (0, NPAD - N))
    b3 = bpad.reshape(GRID, 1, BLK)
    b1r = b1.reshape(1, D)
    W2r = W2.reshape(1, D)
    b2r = b2.reshape(1, 1)

    e1d, den = pl.pallas_call(
        _score_body,
        grid=(GRID,),
        in_specs=[
            pl.BlockSpec((BLK, D), lambda i: (i, 0)),
            pl.BlockSpec((1, 1, BLK), lambda i: (i, 0, 0)),
            pl.BlockSpec((D, D), lambda i: (0, 0)),
            pl.BlockSpec((1, D), lambda i: (0, 0)),
            pl.BlockSpec((1, D), lambda i: (0, 0)),
            pl.BlockSpec((1, 1), lambda i: (0, 0)),
        ],
        out_specs=[
            pl.BlockSpec((BLK,), lambda i: (i,)),
            pl.BlockSpec((NSEG, 128), lambda i: (0, 0)),
        ],
        out_shape=[
            jax.ShapeDtypeStruct((NPAD,), jnp.float32),
            jax.ShapeDtypeStruct((NSEG, 128), jnp.float32),
        ],
    )(x, b3, W1, b1r, W2r, b2r)

    mesh = plsc.VectorSubcoreMesh(core_axis_name="c", subcore_axis_name="s",
                                  num_cores=2, num_subcores=16)
    attn = pl.kernel(
        _attn_body,
        out_type=jax.ShapeDtypeStruct((NPAD,), jnp.float32),
        mesh=mesh,
        scratch_types=[
            pltpu.VMEM((PER_W,), jnp.float32),
            pltpu.VMEM((PER_W,), jnp.int32),
            pltpu.VMEM((PER_W,), jnp.float32),
            pltpu.VMEM((NSEG,), jnp.float32),
        ],
    )(e1d, bpad, den[:, 0])

    out = pl.pallas_call(
        _pool_body,
        grid=(GRID,),
        in_specs=[
            pl.BlockSpec((BLK, D), lambda i: (i, 0)),
            pl.BlockSpec((BLK,), lambda i: (i,)),
            pl.BlockSpec((1, 1, BLK), lambda i: (i, 0, 0)),
        ],
        out_specs=pl.BlockSpec((NSEG, D), lambda i: (0, 0)),
        out_shape=jax.ShapeDtypeStruct((NSEG, D), jnp.float32),
    )(x, attn, b3)
    return out


# speedup vs baseline: 8.4799x; 1.4282x over previous
import jax
import jax.numpy as jnp
from jax import lax
from jax.experimental import pallas as pl
from jax.experimental.pallas import tpu as pltpu
from jax.experimental.pallas import tpu_sc as plsc

N = 50000
D = 256
NSEG = 64

NPAD = 50176
BLK = 1024
GRID = NPAD // BLK

NW = 32
RPW = NSEG // NW


def _fused_body(x_ref, b3_ref, W1_ref, b1_ref, W2_ref, b2_ref, up_ref,
                den_ref):
    i = pl.program_id(0)
    x = x_ref[...]
    h = jnp.tanh(jnp.dot(x, W1_ref[...], preferred_element_type=jnp.float32)
                 + b1_ref[...])
    s = jnp.sum(h * W2_ref[...], axis=1) + b2_ref[0, 0]
    row = jax.lax.broadcasted_iota(jnp.int32, (BLK,), 0) + i * BLK
    e = jnp.where(row < N, jnp.exp(s), 0.0)
    bb = b3_ref[0, 0, :]
    seg = jax.lax.broadcasted_iota(jnp.int32, (NSEG, BLK), 0)
    m = jnp.where(bb[None, :] == seg, e[None, :], 0.0)

    @pl.when(i == 0)
    def _init():
        up_ref[...] = jnp.zeros_like(up_ref)
        den_ref[...] = jnp.zeros_like(den_ref)

    xm = jnp.where(row[:, None] < N, x, 0.0)
    up_ref[...] = up_ref[...] + jnp.dot(
        m, xm, preferred_element_type=jnp.float32)
    den_ref[...] = den_ref[...] + jnp.sum(m, axis=1)[:, None]


def _norm_body(up_hbm, den_hbm, out_hbm, u_v, d_v):
    cid = lax.axis_index("c")
    sid = lax.axis_index("s")
    wid = cid * 16 + sid
    r0 = wid * RPW
    pltpu.sync_copy(up_hbm.at[pl.ds(r0, RPW)], u_v)
    pltpu.sync_copy(den_hbm, d_v.at[pl.ds(0, NSEG)])
    dvec = d_v[pl.ds(r0, 16)]
    rvv = 1.0 / (dvec + 1e-16)
    for r in range(RPW):
        rv = rvv[r]
        for j in range(D // 16):
            sl = pl.ds(j * 16, 16)
            u_v[r, sl] = u_v[r, sl] * rv
    pltpu.sync_copy(u_v, out_hbm.at[pl.ds(r0, RPW)])


def kernel(x, batch, W1, b1, W2, b2):
    batch32 = batch.astype(jnp.int32)
    bpad = jnp.pad(batch32, (0, NPAD - N))
    b3 = bpad.reshape(GRID, 1, BLK)
    b1r = b1.reshape(1, D)
    W2r = W2.reshape(1, D)
    b2r = b2.reshape(1, 1)

    up, den = pl.pallas_call(
        _fused_body,
        grid=(GRID,),
        in_specs=[
            pl.BlockSpec((BLK, D), lambda i: (i, 0)),
            pl.BlockSpec((1, 1, BLK), lambda i: (i, 0, 0)),
            pl.BlockSpec((D, D), lambda i: (0, 0)),
            pl.BlockSpec((1, D), lambda i: (0, 0)),
            pl.BlockSpec((1, D), lambda i: (0, 0)),
            pl.BlockSpec((1, 1), lambda i: (0, 0)),
        ],
        out_specs=[
            pl.BlockSpec((NSEG, D), lambda i: (0, 0)),
            pl.BlockSpec((NSEG, 128), lambda i: (0, 0)),
        ],
        out_shape=[
            jax.ShapeDtypeStruct((NSEG, D), jnp.float32),
            jax.ShapeDtypeStruct((NSEG, 128), jnp.float32),
        ],
    )(x, b3, W1, b1r, W2r, b2r)

    mesh = plsc.VectorSubcoreMesh(core_axis_name="c", subcore_axis_name="s",
                                  num_cores=2, num_subcores=16)
    out = pl.kernel(
        _norm_body,
        out_type=jax.ShapeDtypeStruct((NSEG, D), jnp.float32),
        mesh=mesh,
        scratch_types=[
            pltpu.VMEM((RPW, D), jnp.float32),
            pltpu.VMEM((NSEG + 16,), jnp.float32),
        ],
    )(up, den[:, 0])
    return out


# speedup vs baseline: 10.6454x; 1.2554x over previous
import jax
import jax.numpy as jnp
from jax import lax
from jax.experimental import pallas as pl
from jax.experimental.pallas import tpu as pltpu
from jax.experimental.pallas import tpu_sc as plsc

N = 50000
D = 256
NSEG = 64

NPAD = 50176
BLK = 1792
GRID = NPAD // BLK

NW = 32
RPW = NSEG // NW


def _fused_body(x_ref, b3_ref, W1_ref, b1_ref, W2_ref, b2_ref, up_ref,
                den_ref):
    i = pl.program_id(0)
    x = x_ref[...]
    h = jnp.tanh(jnp.dot(x, W1_ref[...], preferred_element_type=jnp.float32)
                 + b1_ref[...])
    s = jnp.sum(h * W2_ref[...], axis=1) + b2_ref[0, 0]
    row = jax.lax.broadcasted_iota(jnp.int32, (BLK,), 0) + i * BLK
    e = jnp.where(row < N, jnp.exp(s), 0.0)
    bb = b3_ref[0, 0, :]
    seg = jax.lax.broadcasted_iota(jnp.int32, (NSEG, BLK), 0)
    m = jnp.where(bb[None, :] == seg, e[None, :], 0.0)

    @pl.when(i == 0)
    def _init():
        up_ref[...] = jnp.zeros_like(up_ref)
        den_ref[...] = jnp.zeros_like(den_ref)

    xm = jnp.where(row[:, None] < N, x, 0.0)
    up_ref[...] = up_ref[...] + jnp.dot(
        m, xm, preferred_element_type=jnp.float32)
    den_ref[...] = den_ref[...] + jnp.sum(m, axis=1)[:, None]


def _norm_body(up_hbm, den_hbm, out_hbm, u_v, d_v):
    cid = lax.axis_index("c")
    sid = lax.axis_index("s")
    wid = cid * 16 + sid
    r0 = wid * RPW
    pltpu.sync_copy(up_hbm.at[pl.ds(r0, RPW)], u_v)
    pltpu.sync_copy(den_hbm, d_v.at[pl.ds(0, NSEG)])
    dvec = d_v[pl.ds(r0, 16)]
    rvv = 1.0 / (dvec + 1e-16)
    for r in range(RPW):
        rv = rvv[r]
        for j in range(D // 16):
            sl = pl.ds(j * 16, 16)
            u_v[r, sl] = u_v[r, sl] * rv
    pltpu.sync_copy(u_v, out_hbm.at[pl.ds(r0, RPW)])


def kernel(x, batch, W1, b1, W2, b2):
    batch32 = batch.astype(jnp.int32)
    bpad = jnp.pad(batch32, (0, NPAD - N))
    b3 = bpad.reshape(GRID, 1, BLK)
    b1r = b1.reshape(1, D)
    W2r = W2.reshape(1, D)
    b2r = b2.reshape(1, 1)

    up, den = pl.pallas_call(
        _fused_body,
        grid=(GRID,),
        in_specs=[
            pl.BlockSpec((BLK, D), lambda i: (i, 0)),
            pl.BlockSpec((1, 1, BLK), lambda i: (i, 0, 0)),
            pl.BlockSpec((D, D), lambda i: (0, 0)),
            pl.BlockSpec((1, D), lambda i: (0, 0)),
            pl.BlockSpec((1, D), lambda i: (0, 0)),
            pl.BlockSpec((1, 1), lambda i: (0, 0)),
        ],
        out_specs=[
            pl.BlockSpec((NSEG, D), lambda i: (0, 0)),
            pl.BlockSpec((NSEG, 128), lambda i: (0, 0)),
        ],
        out_shape=[
            jax.ShapeDtypeStruct((NSEG, D), jnp.float32),
            jax.ShapeDtypeStruct((NSEG, 128), jnp.float32),
        ],
    )(x, b3, W1, b1r, W2r, b2r)

    mesh = plsc.VectorSubcoreMesh(core_axis_name="c", subcore_axis_name="s",
                                  num_cores=2, num_subcores=16)
    out = pl.kernel(
        _norm_body,
        out_type=jax.ShapeDtypeStruct((NSEG, D), jnp.float32),
        mesh=mesh,
        scratch_types=[
            pltpu.VMEM((RPW, D), jnp.float32),
            pltpu.VMEM((NSEG + 16,), jnp.float32),
        ],
    )(up, den[:, 0])
    return out


# speedup vs baseline: 12.6759x; 1.1907x over previous
import jax
import jax.numpy as jnp
from jax import lax
from jax.experimental import pallas as pl
from jax.experimental.pallas import tpu as pltpu
from jax.experimental.pallas import tpu_sc as plsc

N = 50000
D = 256
NSEG = 64

NPAD = 50176
BLK = 3584
GRID = NPAD // BLK

NW = 32
RPW = NSEG // NW


def _fused_body(x_ref, b3_ref, W1_ref, b1_ref, W2_ref, b2_ref, up_ref,
                den_ref):
    i = pl.program_id(0)
    x = x_ref[...]
    h = jnp.tanh(jnp.dot(x, W1_ref[...], preferred_element_type=jnp.float32)
                 + b1_ref[...])
    s = jnp.sum(h * W2_ref[...], axis=1) + b2_ref[0, 0]
    row = jax.lax.broadcasted_iota(jnp.int32, (BLK,), 0) + i * BLK
    e = jnp.where(row < N, jnp.exp(s), 0.0)
    bb = b3_ref[0, 0, :]
    seg = jax.lax.broadcasted_iota(jnp.int32, (NSEG, BLK), 0)
    m = jnp.where(bb[None, :] == seg, e[None, :], 0.0)

    @pl.when(i == 0)
    def _init():
        up_ref[...] = jnp.zeros_like(up_ref)
        den_ref[...] = jnp.zeros_like(den_ref)

    xm = jnp.where(row[:, None] < N, x, 0.0)
    up_ref[...] = up_ref[...] + jnp.dot(
        m, xm, preferred_element_type=jnp.float32)
    den_ref[...] = den_ref[...] + jnp.sum(m, axis=1)[:, None]


def _norm_body(up_hbm, den_hbm, out_hbm, u_v, d_v):
    cid = lax.axis_index("c")
    sid = lax.axis_index("s")
    wid = cid * 16 + sid
    r0 = wid * RPW
    pltpu.sync_copy(up_hbm.at[pl.ds(r0, RPW)], u_v)
    pltpu.sync_copy(den_hbm, d_v.at[pl.ds(0, NSEG)])
    dvec = d_v[pl.ds(r0, 16)]
    rvv = 1.0 / (dvec + 1e-16)
    for r in range(RPW):
        rv = rvv[r]
        for j in range(D // 16):
            sl = pl.ds(j * 16, 16)
            u_v[r, sl] = u_v[r, sl] * rv
    pltpu.sync_copy(u_v, out_hbm.at[pl.ds(r0, RPW)])


def kernel(x, batch, W1, b1, W2, b2):
    batch32 = batch.astype(jnp.int32)
    bpad = jnp.pad(batch32, (0, NPAD - N))
    b3 = bpad.reshape(GRID, 1, BLK)
    b1r = b1.reshape(1, D)
    W2r = W2.reshape(1, D)
    b2r = b2.reshape(1, 1)

    up, den = pl.pallas_call(
        _fused_body,
        grid=(GRID,),
        in_specs=[
            pl.BlockSpec((BLK, D), lambda i: (i, 0)),
            pl.BlockSpec((1, 1, BLK), lambda i: (i, 0, 0)),
            pl.BlockSpec((D, D), lambda i: (0, 0)),
            pl.BlockSpec((1, D), lambda i: (0, 0)),
            pl.BlockSpec((1, D), lambda i: (0, 0)),
            pl.BlockSpec((1, 1), lambda i: (0, 0)),
        ],
        out_specs=[
            pl.BlockSpec((NSEG, D), lambda i: (0, 0)),
            pl.BlockSpec((NSEG, 128), lambda i: (0, 0)),
        ],
        out_shape=[
            jax.ShapeDtypeStruct((NSEG, D), jnp.float32),
            jax.ShapeDtypeStruct((NSEG, 128), jnp.float32),
        ],
    )(x, b3, W1, b1r, W2r, b2r)

    mesh = plsc.VectorSubcoreMesh(core_axis_name="c", subcore_axis_name="s",
                                  num_cores=2, num_subcores=16)
    out = pl.kernel(
        _norm_body,
        out_type=jax.ShapeDtypeStruct((NSEG, D), jnp.float32),
        mesh=mesh,
        scratch_types=[
            pltpu.VMEM((RPW, D), jnp.float32),
            pltpu.VMEM((NSEG + 16,), jnp.float32),
        ],
    )(up, den[:, 0])
    return out


# speedup vs baseline: 14.1189x; 1.1138x over previous
import jax
import jax.numpy as jnp
from jax import lax
from jax.experimental import pallas as pl
from jax.experimental.pallas import tpu as pltpu
from jax.experimental.pallas import tpu_sc as plsc

N = 50000
D = 256
NSEG = 64

NPAD = 50176
BLK = 7168
GRID = NPAD // BLK

NW = 32
RPW = NSEG // NW


def _fused_body(x_ref, b3_ref, W1_ref, b1_ref, W2_ref, b2_ref, up_ref,
                den_ref):
    i = pl.program_id(0)
    x = x_ref[...]
    h = jnp.tanh(jnp.dot(x, W1_ref[...], preferred_element_type=jnp.float32)
                 + b1_ref[...])
    s = jnp.sum(h * W2_ref[...], axis=1) + b2_ref[0, 0]
    e = jnp.exp(s)
    bb = b3_ref[0, 0, :]
    seg = jax.lax.broadcasted_iota(jnp.int32, (NSEG, BLK), 0)

    @pl.when(i == 0)
    def _init():
        up_ref[...] = jnp.zeros_like(up_ref)
        den_ref[...] = jnp.zeros_like(den_ref)

    @pl.when(i < GRID - 1)
    def _full():
        m = jnp.where(bb[None, :] == seg, e[None, :], 0.0)
        up_ref[...] = up_ref[...] + jnp.dot(
            m, x, preferred_element_type=jnp.float32)
        den_ref[...] = den_ref[...] + jnp.sum(m, axis=1)[:, None]

    @pl.when(i == GRID - 1)
    def _tail():
        row = jax.lax.broadcasted_iota(jnp.int32, (BLK,), 0) + i * BLK
        em = jnp.where(row < N, e, 0.0)
        m = jnp.where(bb[None, :] == seg, em[None, :], 0.0)
        xm = jnp.where(row[:, None] < N, x, 0.0)
        up_ref[...] = up_ref[...] + jnp.dot(
            m, xm, preferred_element_type=jnp.float32)
        den_ref[...] = den_ref[...] + jnp.sum(m, axis=1)[:, None]


def _norm_body(up_hbm, den_hbm, out_hbm, u_v, d_v):
    cid = lax.axis_index("c")
    sid = lax.axis_index("s")
    wid = cid * 16 + sid
    r0 = wid * RPW
    pltpu.sync_copy(up_hbm.at[pl.ds(r0, RPW)], u_v)
    pltpu.sync_copy(den_hbm, d_v.at[pl.ds(0, NSEG)])
    dvec = d_v[pl.ds(r0, 16)]
    rvv = 1.0 / (dvec + 1e-16)
    for r in range(RPW):
        rv = rvv[r]
        for j in range(D // 16):
            sl = pl.ds(j * 16, 16)
            u_v[r, sl] = u_v[r, sl] * rv
    pltpu.sync_copy(u_v, out_hbm.at[pl.ds(r0, RPW)])


def kernel(x, batch, W1, b1, W2, b2):
    batch32 = batch.astype(jnp.int32)
    bpad = jnp.pad(batch32, (0, NPAD - N))
    b3 = bpad.reshape(GRID, 1, BLK)
    b1r = b1.reshape(1, D)
    W2r = W2.reshape(1, D)
    b2r = b2.reshape(1, 1)

    up, den = pl.pallas_call(
        _fused_body,
        grid=(GRID,),
        in_specs=[
            pl.BlockSpec((BLK, D), lambda i: (i, 0)),
            pl.BlockSpec((1, 1, BLK), lambda i: (i, 0, 0)),
            pl.BlockSpec((D, D), lambda i: (0, 0)),
            pl.BlockSpec((1, D), lambda i: (0, 0)),
            pl.BlockSpec((1, D), lambda i: (0, 0)),
            pl.BlockSpec((1, 1), lambda i: (0, 0)),
        ],
        out_specs=[
            pl.BlockSpec((NSEG, D), lambda i: (0, 0)),
            pl.BlockSpec((NSEG, 128), lambda i: (0, 0)),
        ],
        out_shape=[
            jax.ShapeDtypeStruct((NSEG, D), jnp.float32),
            jax.ShapeDtypeStruct((NSEG, 128), jnp.float32),
        ],
    )(x, b3, W1, b1r, W2r, b2r)

    mesh = plsc.VectorSubcoreMesh(core_axis_name="c", subcore_axis_name="s",
                                  num_cores=2, num_subcores=16)
    out = pl.kernel(
        _norm_body,
        out_type=jax.ShapeDtypeStruct((NSEG, D), jnp.float32),
        mesh=mesh,
        scratch_types=[
            pltpu.VMEM((RPW, D), jnp.float32),
            pltpu.VMEM((NSEG + 16,), jnp.float32),
        ],
    )(up, den[:, 0])
    return out


# speedup vs baseline: 14.1590x; 1.0028x over previous
import jax
import jax.numpy as jnp
from jax import lax
from jax.experimental import pallas as pl
from jax.experimental.pallas import tpu as pltpu
from jax.experimental.pallas import tpu_sc as plsc

N = 50000
D = 256
NSEG = 64

NPAD = 50176
BLK = 12544
GRID = NPAD // BLK

NW = 32
RPW = NSEG // NW


def _fused_body(x_ref, b3_ref, W1_ref, b1_ref, W2_ref, b2_ref, up_ref,
                den_ref):
    i = pl.program_id(0)
    x = x_ref[...]
    h = jnp.tanh(jnp.dot(x, W1_ref[...], preferred_element_type=jnp.float32)
                 + b1_ref[...])
    s = jnp.sum(h * W2_ref[...], axis=1) + b2_ref[0, 0]
    e = jnp.exp(s)
    bb = b3_ref[0, 0, :]
    seg = jax.lax.broadcasted_iota(jnp.int32, (NSEG, BLK), 0)

    @pl.when(i == 0)
    def _init():
        up_ref[...] = jnp.zeros_like(up_ref)
        den_ref[...] = jnp.zeros_like(den_ref)

    @pl.when(i < GRID - 1)
    def _full():
        m = jnp.where(bb[None, :] == seg, e[None, :], 0.0)
        up_ref[...] = up_ref[...] + jnp.dot(
            m, x, preferred_element_type=jnp.float32)
        den_ref[...] = den_ref[...] + jnp.sum(m, axis=1)[:, None]

    @pl.when(i == GRID - 1)
    def _tail():
        row = jax.lax.broadcasted_iota(jnp.int32, (BLK,), 0) + i * BLK
        em = jnp.where(row < N, e, 0.0)
        m = jnp.where(bb[None, :] == seg, em[None, :], 0.0)
        xm = jnp.where(row[:, None] < N, x, 0.0)
        up_ref[...] = up_ref[...] + jnp.dot(
            m, xm, preferred_element_type=jnp.float32)
        den_ref[...] = den_ref[...] + jnp.sum(m, axis=1)[:, None]


def _norm_body(up_hbm, den_hbm, out_hbm, u_v, d_v):
    cid = lax.axis_index("c")
    sid = lax.axis_index("s")
    wid = cid * 16 + sid
    r0 = wid * RPW
    pltpu.sync_copy(up_hbm.at[pl.ds(r0, RPW)], u_v)
    pltpu.sync_copy(den_hbm, d_v.at[pl.ds(0, NSEG)])
    dvec = d_v[pl.ds(r0, 16)]
    rvv = 1.0 / (dvec + 1e-16)
    for r in range(RPW):
        rv = rvv[r]
        for j in range(D // 16):
            sl = pl.ds(j * 16, 16)
            u_v[r, sl] = u_v[r, sl] * rv
    pltpu.sync_copy(u_v, out_hbm.at[pl.ds(r0, RPW)])


def kernel(x, batch, W1, b1, W2, b2):
    batch32 = batch.astype(jnp.int32)
    bpad = jnp.pad(batch32, (0, NPAD - N))
    b3 = bpad.reshape(GRID, 1, BLK)
    b1r = b1.reshape(1, D)
    W2r = W2.reshape(1, D)
    b2r = b2.reshape(1, 1)

    up, den = pl.pallas_call(
        _fused_body,
        grid=(GRID,),
        in_specs=[
            pl.BlockSpec((BLK, D), lambda i: (i, 0)),
            pl.BlockSpec((1, 1, BLK), lambda i: (i, 0, 0)),
            pl.BlockSpec((D, D), lambda i: (0, 0)),
            pl.BlockSpec((1, D), lambda i: (0, 0)),
            pl.BlockSpec((1, D), lambda i: (0, 0)),
            pl.BlockSpec((1, 1), lambda i: (0, 0)),
        ],
        out_specs=[
            pl.BlockSpec((NSEG, D), lambda i: (0, 0)),
            pl.BlockSpec((NSEG, 128), lambda i: (0, 0)),
        ],
        out_shape=[
            jax.ShapeDtypeStruct((NSEG, D), jnp.float32),
            jax.ShapeDtypeStruct((NSEG, 128), jnp.float32),
        ],
    )(x, b3, W1, b1r, W2r, b2r)

    mesh = plsc.VectorSubcoreMesh(core_axis_name="c", subcore_axis_name="s",
                                  num_cores=2, num_subcores=16)
    out = pl.kernel(
        _norm_body,
        out_type=jax.ShapeDtypeStruct((NSEG, D), jnp.float32),
        mesh=mesh,
        scratch_types=[
            pltpu.VMEM((RPW, D), jnp.float32),
            pltpu.VMEM((NSEG + 16,), jnp.float32),
        ],
    )(up, den[:, 0])
    return out
